# baseline (speedup 1.0000x reference)
import dataclasses
import functools

import jax
import jax.numpy as jnp
from jax import lax
from jax.experimental import pallas as pl
from jax.experimental.pallas import tpu as pltpu
from jax.experimental.pallas import tpu_sc as plsc

NC = 2
NS = 16
NW = NC * NS
L16 = 16

HID = 64
KV = HID // L16

C = 128
NT = 6


def _rsqrt1(x):
    i = lax.bitcast_convert_type(x, jnp.int32)
    i = jnp.int32(0x5F375A86) - lax.shift_right_arithmetic(i, 1)
    y = lax.bitcast_convert_type(i, jnp.float32)
    y = y * (1.5 - 0.5 * x * y * y)
    return y * (1.5 - 0.5 * x * y * y)


def _bsum(v):
    iota = lax.iota(jnp.int32, L16)
    for kbit in (8, 4, 2, 1):
        idx = jnp.bitwise_xor(iota, jnp.int32(kbit))
        v = v + v.at[idx].get(mode="promise_in_bounds")
    return v


@functools.partial(jax.jit, static_argnames=("n_tok",))
def _embed_ln(n_tok, ids6, wtab, dtab, ptab, stab, gamma, beta):
    tok_w = n_tok // NW
    nchunk = tok_w // C
    assert nchunk % 3 == 2 and nchunk >= 5
    rows_w = nchunk
    n_rows = n_tok // C
    mesh = plsc.VectorSubcoreMesh(core_axis_name="c", subcore_axis_name="s")
    cp = pltpu.CompilerParams()
    if "needs_layout_passes" in pltpu.CompilerParams.__dataclass_fields__:
        cp = dataclasses.replace(cp, needs_layout_passes=False)
    if "use_tc_tiling_on_sc" in pltpu.CompilerParams.__dataclass_fields__:
        cp = dataclasses.replace(cp, use_tc_tiling_on_sc=False)

    @functools.partial(
        pl.kernel,
        compiler_params=cp,
        out_type=jax.ShapeDtypeStruct((n_rows, C, HID), jnp.float32),
        mesh=mesh,
        scratch_types=[
            pltpu.VMEM((NT * rows_w, C), jnp.int32),
            pltpu.VMEM((C, HID), jnp.float32),
            pltpu.VMEM((C, HID), jnp.float32),
            pltpu.VMEM((C, HID), jnp.float32),
            pltpu.VMEM((HID,), jnp.float32),
            pltpu.VMEM((HID,), jnp.float32),
            pltpu.SemaphoreType.DMA,
            pltpu.SemaphoreType.DMA,
            pltpu.SemaphoreType.DMA,
            pltpu.SemaphoreType.DMA,
            pltpu.SemaphoreType.DMA,
            pltpu.SemaphoreType.DMA,
            pltpu.SemaphoreType.DMA,
            pltpu.SemaphoreType.DMA,
            pltpu.SemaphoreType.DMA,
        ],
    )
    def k(ids6_h, wtab_h, dtab_h, ptab_h, stab_h, gamma_h, beta_h, out_h,
          ids_v, acc0, acc1, acc2, g_v, b_v,
          sa0, sa1, sa2, sm0, sm1, sm2, so0, so1, so2):
        wid = lax.axis_index("s") * NC + lax.axis_index("c")
        row0 = wid * rows_w

        pltpu.sync_copy(gamma_h, g_v)
        pltpu.sync_copy(beta_h, b_v)
        pltpu.sync_copy(ids6_h.at[wid], ids_v)

        acc = (acc0, acc1, acc2)
        sem_a = (sa0, sa1, sa2)
        sem_m = (sm0, sm1, sm2)
        sem_o = (so0, so1, so2)
        AGE, WORD, BMI, CYC, SEG, POS = range(NT)

        def idx(k, g):
            return ids_v.at[k * rows_w + g]

        def issue_age(g, p):
            pltpu.async_copy(dtab_h.at[idx(AGE, g)], acc[p], sem_a[p])

        def issue_adds(g, p):
            pltpu.async_copy(wtab_h.at[idx(WORD, g)], acc[p], sem_m[p],
                             add=True)
            pltpu.async_copy(dtab_h.at[idx(BMI, g)], acc[p], sem_m[p],
                             add=True)
            pltpu.async_copy(dtab_h.at[idx(CYC, g)], acc[p], sem_m[p],
                             add=True)
            pltpu.async_copy(stab_h.at[idx(SEG, g)], acc[p], sem_m[p],
                             add=True)
            pltpu.async_copy(ptab_h.at[idx(POS, g)], acc[p], sem_m[p],
                             add=True)

        def wait_age(g, p):
            pltpu.make_async_copy(
                dtab_h.at[idx(AGE, g)], acc[p], sem_a[p]).wait()

        def wait_adds(g, p):
            pltpu.make_async_copy(
                wtab_h.at[idx(WORD, g)], acc[p], sem_m[p]).wait()
            pltpu.make_async_copy(
                dtab_h.at[idx(BMI, g)], acc[p], sem_m[p]).wait()
            pltpu.make_async_copy(
                dtab_h.at[idx(CYC, g)], acc[p], sem_m[p]).wait()
            pltpu.make_async_copy(
                stab_h.at[idx(SEG, g)], acc[p], sem_m[p]).wait()
            pltpu.make_async_copy(
                ptab_h.at[idx(POS, g)], acc[p], sem_m[p]).wait()

        def issue_flush(g, p):
            pltpu.async_copy(acc[p], out_h.at[row0 + g], sem_o[p])

        def wait_flush(p):
            pltpu.make_async_copy(acc[p], out_h.at[row0], sem_o[p]).wait()

        def compute(p):
            ab = acc[p]

            @pl.loop(0, C // L16)
            def _grp(gg):
                s = gg * L16
                gvec = [g_v[pl.ds(kk * L16, L16)] for kk in range(KV)]
                bvec = [b_v[pl.ds(kk * L16, L16)] for kk in range(KV)]
                for j in range(L16):
                    t = s + j
                    a = [ab[t, pl.ds(kk * L16, L16)] for kk in range(KV)]
                    s1 = (a[0] + a[1]) + (a[2] + a[3])
                    sq = ((a[0] * a[0] + a[1] * a[1])
                          + (a[2] * a[2] + a[3] * a[3]))
                    mvec = _bsum(s1) * (1.0 / HID)
                    ex2 = _bsum(sq) * (1.0 / HID)
                    var = ex2 - mvec * mvec
                    rstd = _rsqrt1(var + 1e-12)
                    for kk in range(KV):
                        ab[t, pl.ds(kk * L16, L16)] = (
                            (a[kk] - mvec) * (rstd * gvec[kk]) + bvec[kk])

        def do_chunk(g, p, p1, p2, next2, nxt):
            if next2:
                @pl.when(g >= 1)
                def _():
                    wait_flush(p2)
                issue_age(g + 2, p2)
            if nxt:
                wait_age(g + 1, p1)
                issue_adds(g + 1, p1)
            wait_adds(g, p)
            compute(p)
            issue_flush(g, p)

        issue_age(0, 0)
        wait_age(0, 0)
        issue_adds(0, 0)
        issue_age(1, 1)

        @pl.loop(0, (nchunk - 2) // 3)
        def _trip(i):
            g = i * 3
            do_chunk(g, 0, 1, 2, True, True)
            do_chunk(g + 1, 1, 2, 0, True, True)
            do_chunk(g + 2, 2, 0, 1, True, True)

        do_chunk(nchunk - 2, 0, 1, 2, False, True)
        do_chunk(nchunk - 1, 1, 2, 0, False, False)

        wait_flush(2)
        wait_flush(0)
        wait_flush(1)

    return k(ids6, wtab, dtab, ptab, stab, gamma, beta)


def kernel(word_ids, age_ids, bmi_ids, cycle_len_ids, seg_ids, posi_ids,
           word_table, demo_table, posi_table, seg_table, ln_gamma, ln_beta):
    b, l = word_ids.shape
    n_tok = b * l
    rows_w = n_tok // (NW * C)
    as_w = lambda x: x.reshape(NW, rows_w, C).astype(jnp.int32)
    ids6 = jnp.stack(
        [as_w(age_ids), as_w(word_ids), as_w(bmi_ids),
         as_w(cycle_len_ids), as_w(seg_ids), as_w(posi_ids)],
        axis=1).reshape(NW, NT * rows_w, C)
    out = _embed_ln(
        n_tok, ids6,
        word_table.astype(jnp.float32),
        demo_table.astype(jnp.float32),
        posi_table.astype(jnp.float32),
        seg_table.astype(jnp.float32),
        ln_gamma.astype(jnp.float32), ln_beta.astype(jnp.float32),
    )
    return out.reshape(b, l, HID)


# speedup vs baseline: 7.5797x; 7.5797x over previous
import dataclasses
import functools

import jax
import jax.numpy as jnp
from jax import lax
from jax.experimental import pallas as pl
from jax.experimental.pallas import tpu as pltpu
from jax.experimental.pallas import tpu_sc as plsc

NC = 2
NS = 16
NW = NC * NS
L16 = 16

HID = 64
KV = HID // L16

DEMO_VOCAB = 128
MAX_POS = 512

C = 128
NS5 = 5


def _rsqrt2(x):
    i = lax.bitcast_convert_type(x, jnp.int32)
    i = jnp.int32(0x5F375A86) - lax.shift_right_arithmetic(i, 1)
    y = lax.bitcast_convert_type(i, jnp.float32)
    y = y * (1.5 - 0.5 * x * y * y)
    return y * (1.5 - 0.5 * x * y * y)


def _bsum(v):
    iota = lax.iota(jnp.int32, L16)
    for kbit in (8, 4, 2, 1):
        idx = jnp.bitwise_xor(iota, jnp.int32(kbit))
        v = v + v.at[idx].get(mode="promise_in_bounds")
    return v


def _bcast(vec, j):
    return vec.at[jnp.full((L16,), j, jnp.int32)].get(
        mode="promise_in_bounds")


@functools.partial(jax.jit, static_argnames=("n_tok",))
def _embed_ln(n_tok, idw2, ids5, wtab, dtab_f, ptab_f, stab_f, gamma, beta):
    tok_w = n_tok // NW
    nchunk = tok_w // C
    assert nchunk % 3 == 2 and nchunk >= 5
    rows_w = nchunk
    n_rows = n_tok // C
    mesh = plsc.VectorSubcoreMesh(core_axis_name="c", subcore_axis_name="s")
    cp = pltpu.CompilerParams()
    if "needs_layout_passes" in pltpu.CompilerParams.__dataclass_fields__:
        cp = dataclasses.replace(cp, needs_layout_passes=False)
    if "use_tc_tiling_on_sc" in pltpu.CompilerParams.__dataclass_fields__:
        cp = dataclasses.replace(cp, use_tc_tiling_on_sc=False)

    @functools.partial(
        pl.kernel,
        compiler_params=cp,
        out_type=jax.ShapeDtypeStruct((n_rows, C, HID), jnp.float32),
        mesh=mesh,
        scratch_types=[
            pltpu.VMEM((2 * rows_w, C // 2), jnp.int32),
            pltpu.VMEM((NS5 * rows_w, C), jnp.int32),
            pltpu.VMEM((C, HID), jnp.float32),
            pltpu.VMEM((C, HID), jnp.float32),
            pltpu.VMEM((C, HID), jnp.float32),
            pltpu.VMEM((DEMO_VOCAB * HID,), jnp.float32),
            pltpu.VMEM((MAX_POS * HID,), jnp.float32),
            pltpu.VMEM((2 * HID,), jnp.float32),
            pltpu.VMEM((HID,), jnp.float32),
            pltpu.VMEM((HID,), jnp.float32),
            pltpu.SemaphoreType.DMA,
            pltpu.SemaphoreType.DMA,
            pltpu.SemaphoreType.DMA,
            pltpu.SemaphoreType.DMA,
            pltpu.SemaphoreType.DMA,
            pltpu.SemaphoreType.DMA,
        ],
    )
    def k(idw2_h, ids5_h, wtab_h, dtab_h, ptab_h, stab_h, gamma_h, beta_h,
          out_h,
          idwb, idsb, wr0, wr1, wr2, dtab_v, ptab_v, stab_v, g_v, b_v,
          sg0, sg1, sg2, so0, so1, so2):
        wid = lax.axis_index("s") * NC + lax.axis_index("c")
        row0 = wid * rows_w

        pltpu.sync_copy(dtab_h, dtab_v)
        pltpu.sync_copy(ptab_h, ptab_v)
        pltpu.sync_copy(stab_h, stab_v)
        pltpu.sync_copy(gamma_h, g_v)
        pltpu.sync_copy(beta_h, b_v)
        pltpu.sync_copy(idw2_h.at[wid], idwb)
        pltpu.sync_copy(ids5_h.at[wid], idsb)

        wrs = (wr0, wr1, wr2)
        sem_g = (sg0, sg1, sg2)
        sem_o = (so0, so1, so2)
        AGE, BMI, CYC, SEG, POS = range(NS5)
        H2 = C // 2

        def issue_word(g, p):
            pltpu.async_copy(
                wtab_h.at[idwb.at[2 * g]],
                wrs[p].at[pl.ds(0, H2)], sem_g[p])
            pltpu.async_copy(
                wtab_h.at[idwb.at[2 * g + 1]],
                wrs[p].at[pl.ds(H2, H2)], sem_g[p])

        def wait_word(g, p):
            pltpu.make_async_copy(
                wtab_h.at[idwb.at[2 * g]],
                wrs[p].at[pl.ds(0, H2)], sem_g[p]).wait()
            pltpu.make_async_copy(
                wtab_h.at[idwb.at[2 * g + 1]],
                wrs[p].at[pl.ds(H2, H2)], sem_g[p]).wait()

        def issue_flush(g, p):
            pltpu.async_copy(wrs[p], out_h.at[row0 + g], sem_o[p])

        def wait_flush(p):
            pltpu.make_async_copy(wrs[p], out_h.at[row0], sem_o[p]).wait()

        def compute(g, p):
            wr = wrs[p]
            iota = lax.iota(jnp.int32, L16)
            cvec = [kk * L16 + iota for kk in range(KV)]

            @pl.loop(0, C // L16)
            def _grp(gg):
                s = gg * L16
                gvec = [g_v[pl.ds(kk * L16, L16)] for kk in range(KV)]
                bvec = [b_v[pl.ds(kk * L16, L16)] for kk in range(KV)]
                seg0 = [stab_v[pl.ds(kk * L16, L16)] for kk in range(KV)]
                segd = [stab_v[pl.ds(HID + kk * L16, L16)] - seg0[kk]
                        for kk in range(KV)]
                av = idsb[AGE * rows_w + g, pl.ds(s, L16)]
                bv = idsb[BMI * rows_w + g, pl.ds(s, L16)]
                cv = idsb[CYC * rows_w + g, pl.ds(s, L16)]
                sv = idsb[SEG * rows_w + g, pl.ds(s, L16)]
                pv = idsb[POS * rows_w + g, pl.ds(s, L16)]
                svf = sv.astype(jnp.float32)

                def grow(tab_v, idv, j):
                    base = _bcast(idv, j) * HID
                    return [plsc.load_gather(tab_v, [base + cvec[kk]])
                            for kk in range(KV)]

                for j in range(L16):
                    t = s + j
                    ar = grow(dtab_v, av, j)
                    br = grow(dtab_v, bv, j)
                    cr = grow(dtab_v, cv, j)
                    pr = grow(ptab_v, pv, j)
                    sf = _bcast(svf, j)

                    acc = []
                    for kk in range(KV):
                        v = ((wr[t, pl.ds(kk * L16, L16)] + ar[kk])
                             + (br[kk] + cr[kk])
                             + (pr[kk] + (seg0[kk] + sf * segd[kk])))
                        acc.append(v)

                    s1 = (acc[0] + acc[1]) + (acc[2] + acc[3])
                    sq = ((acc[0] * acc[0] + acc[1] * acc[1])
                          + (acc[2] * acc[2] + acc[3] * acc[3]))
                    mvec = _bsum(s1) * (1.0 / HID)
                    ex2 = _bsum(sq) * (1.0 / HID)
                    var = ex2 - mvec * mvec
                    rstd = _rsqrt2(var + 1e-12)
                    for kk in range(KV):
                        wr[t, pl.ds(kk * L16, L16)] = (
                            (acc[kk] - mvec) * (rstd * gvec[kk]) + bvec[kk])

        def do_chunk(g, p, p2, steady):
            if steady:
                @pl.when(g >= 1)
                def _():
                    wait_flush(p2)
                issue_word(g + 2, p2)
            wait_word(g, p)
            compute(g, p)
            issue_flush(g, p)

        issue_word(0, 0)
        issue_word(1, 1)

        @pl.loop(0, (nchunk - 2) // 3)
        def _trip(i):
            g = i * 3
            do_chunk(g, 0, 2, True)
            do_chunk(g + 1, 1, 0, True)
            do_chunk(g + 2, 2, 1, True)

        do_chunk(nchunk - 2, 0, 2, False)
        do_chunk(nchunk - 1, 1, 2, False)

        wait_flush(2)
        wait_flush(0)
        wait_flush(1)

    return k(idw2, ids5, wtab, dtab_f, ptab_f, stab_f, gamma, beta)


def kernel(word_ids, age_ids, bmi_ids, cycle_len_ids, seg_ids, posi_ids,
           word_table, demo_table, posi_table, seg_table, ln_gamma, ln_beta):
    b, l = word_ids.shape
    n_tok = b * l
    rows_w = n_tok // (NW * C)
    idw2 = word_ids.reshape(NW, 2 * rows_w, C // 2).astype(jnp.int32)
    as_w = lambda x: x.reshape(NW, rows_w, C).astype(jnp.int32)
    ids5 = jnp.stack(
        [as_w(age_ids), as_w(bmi_ids), as_w(cycle_len_ids),
         as_w(seg_ids), as_w(posi_ids)],
        axis=1).reshape(NW, NS5 * rows_w, C)
    out = _embed_ln(
        n_tok, idw2, ids5,
        word_table.astype(jnp.float32),
        demo_table.astype(jnp.float32).reshape(-1),
        posi_table.astype(jnp.float32).reshape(-1),
        seg_table.astype(jnp.float32).reshape(-1),
        ln_gamma.astype(jnp.float32), ln_beta.astype(jnp.float32),
    )
    return out.reshape(b, l, HID)


# speedup vs baseline: 9.9385x; 1.3112x over previous
import dataclasses
import functools

import jax
import jax.numpy as jnp
from jax import lax
from jax.experimental import pallas as pl
from jax.experimental.pallas import tpu as pltpu
from jax.experimental.pallas import tpu_sc as plsc

NC = 2
NS = 16
NW = NC * NS
L16 = 16

HID = 64
KV = HID // L16

DEMO_VOCAB = 128
MAX_POS = 512

C = 128
NS5 = 5
NB = 4


def _rsqrt(x):
    i = lax.bitcast_convert_type(x, jnp.int32)
    i = jnp.int32(0x5F375A86) - lax.shift_right_arithmetic(i, 1)
    y = lax.bitcast_convert_type(i, jnp.float32)
    return y * (1.5 - 0.5 * x * y * y)


def _xorp(v, iota, kbit):
    return v.at[jnp.bitwise_xor(iota, jnp.int32(kbit))].get(
        mode="promise_in_bounds")


def _bcast(vec, j):
    return vec.at[jnp.full((L16,), j, jnp.int32)].get(
        mode="promise_in_bounds")


@functools.partial(jax.jit, static_argnames=("n_tok",))
def _embed_ln(n_tok, idw2, ids5, wtab, dtab_f, ptab_f, stab_f, gamma, beta):
    tok_w = n_tok // NW
    nchunk = tok_w // C
    assert nchunk % 2 == 0 and nchunk >= NB + 2
    rows_w = nchunk
    n_rows = n_tok // C
    mesh = plsc.VectorSubcoreMesh(core_axis_name="c", subcore_axis_name="s")
    cp = pltpu.CompilerParams()
    if "needs_layout_passes" in pltpu.CompilerParams.__dataclass_fields__:
        cp = dataclasses.replace(cp, needs_layout_passes=False)
    if "use_tc_tiling_on_sc" in pltpu.CompilerParams.__dataclass_fields__:
        cp = dataclasses.replace(cp, use_tc_tiling_on_sc=False)

    @functools.partial(
        pl.kernel,
        compiler_params=cp,
        out_type=jax.ShapeDtypeStruct((n_rows, C, HID), jnp.float32),
        mesh=mesh,
        scratch_types=[
            pltpu.VMEM((2 * rows_w, C // 2), jnp.int32),
            pltpu.VMEM((NS5 * rows_w, C), jnp.int32),
            pltpu.VMEM((C, HID), jnp.float32),
            pltpu.VMEM((C, HID), jnp.float32),
            pltpu.VMEM((C, HID), jnp.float32),
            pltpu.VMEM((C, HID), jnp.float32),
            pltpu.VMEM((DEMO_VOCAB * HID,), jnp.float32),
            pltpu.VMEM((MAX_POS * HID,), jnp.float32),
            pltpu.VMEM((2 * HID,), jnp.float32),
            pltpu.VMEM((HID,), jnp.float32),
            pltpu.VMEM((HID,), jnp.float32),
            pltpu.SemaphoreType.DMA,
            pltpu.SemaphoreType.DMA,
            pltpu.SemaphoreType.DMA,
            pltpu.SemaphoreType.DMA,
            pltpu.SemaphoreType.DMA,
            pltpu.SemaphoreType.DMA,
            pltpu.SemaphoreType.DMA,
            pltpu.SemaphoreType.DMA,
        ],
    )
    def k(idw2_h, ids5_h, wtab_h, dtab_h, ptab_h, stab_h, gamma_h, beta_h,
          out_h,
          idwb, idsb, wr0, wr1, wr2, wr3, dtab_v, ptab_v, stab_v, g_v, b_v,
          sg0, sg1, sg2, sg3, so0, so1, so2, so3):
        wid = lax.axis_index("s") * NC + lax.axis_index("c")
        row0 = wid * rows_w

        pltpu.sync_copy(dtab_h, dtab_v)
        pltpu.sync_copy(ptab_h, ptab_v)
        pltpu.sync_copy(stab_h, stab_v)
        pltpu.sync_copy(gamma_h, g_v)
        pltpu.sync_copy(beta_h, b_v)
        pltpu.sync_copy(idw2_h.at[wid], idwb)
        pltpu.sync_copy(ids5_h.at[wid], idsb)

        wrs = (wr0, wr1, wr2, wr3)
        sem_g = (sg0, sg1, sg2, sg3)
        sem_o = (so0, so1, so2, so3)
        AGE, BMI, CYC, SEG, POS = range(NS5)
        H2 = C // 2

        def issue_word(g, p):
            pltpu.async_copy(
                wtab_h.at[idwb.at[2 * g]],
                wrs[p].at[pl.ds(0, H2)], sem_g[p])
            pltpu.async_copy(
                wtab_h.at[idwb.at[2 * g + 1]],
                wrs[p].at[pl.ds(H2, H2)], sem_g[p])

        def wait_word(g, p):
            pltpu.make_async_copy(
                wtab_h.at[idwb.at[2 * g]],
                wrs[p].at[pl.ds(0, H2)], sem_g[p]).wait()
            pltpu.make_async_copy(
                wtab_h.at[idwb.at[2 * g + 1]],
                wrs[p].at[pl.ds(H2, H2)], sem_g[p]).wait()

        def issue_flush(g, p):
            pltpu.async_copy(wrs[p], out_h.at[row0 + g], sem_o[p])

        def wait_flush(p):
            pltpu.make_async_copy(wrs[p], out_h.at[row0], sem_o[p]).wait()

        def compute(g, p):
            wr = wrs[p]
            iota = lax.iota(jnp.int32, L16)
            cvec = [kk * L16 + iota for kk in range(KV)]
            lo8 = iota < 8
            gvec = [g_v[pl.ds(kk * L16, L16)] for kk in range(KV)]
            bvec = [b_v[pl.ds(kk * L16, L16)] for kk in range(KV)]
            seg0 = [stab_v[pl.ds(kk * L16, L16)] for kk in range(KV)]
            segd = [stab_v[pl.ds(HID + kk * L16, L16)] - seg0[kk]
                    for kk in range(KV)]

            def grow(tab_v, idv, j):
                base = _bcast(idv, j)
                return [plsc.load_gather(tab_v, [base + cvec[kk]])
                        for kk in range(KV)]

            def embed(av, bv, cv, pv, svf, t, j):
                ar = grow(dtab_v, av, j)
                br = grow(dtab_v, bv, j)
                cr = grow(dtab_v, cv, j)
                pr = grow(ptab_v, pv, j)
                sf = _bcast(svf, j)
                acc = []
                for kk in range(KV):
                    v = ((wr[t, pl.ds(kk * L16, L16)] + ar[kk])
                         + (br[kk] + cr[kk])
                         + (pr[kk] + (seg0[kk] + sf * segd[kk])))
                    acc.append(v)
                s1 = (acc[0] + acc[1]) + (acc[2] + acc[3])
                sq = ((acc[0] * acc[0] + acc[1] * acc[1])
                      + (acc[2] * acc[2] + acc[3] * acc[3]))
                return acc, s1, sq

            def fold2(xa, xb):
                m = jnp.where(lo8, xa + _xorp(xa, iota, 8),
                              xb + _xorp(xb, iota, 8))
                for kbit in (4, 2, 1):
                    m = m + _xorp(m, iota, kbit)
                return m

            @pl.loop(0, C // L16)
            def _grp(gg):
                s = gg * L16
                av = idsb[AGE * rows_w + g, pl.ds(s, L16)]
                bv = idsb[BMI * rows_w + g, pl.ds(s, L16)]
                cv = idsb[CYC * rows_w + g, pl.ds(s, L16)]
                sv = idsb[SEG * rows_w + g, pl.ds(s, L16)]
                pv = idsb[POS * rows_w + g, pl.ds(s, L16)]
                svf = sv.astype(jnp.float32)

                for j2 in range(L16 // 2):
                    ta, tb = s + 2 * j2, s + 2 * j2 + 1
                    acc_a, s1a, sqa = embed(av, bv, cv, pv, svf, ta, 2 * j2)
                    acc_b, s1b, sqb = embed(av, bv, cv, pv, svf, tb,
                                            2 * j2 + 1)
                    su = fold2(s1a, s1b)
                    qu = fold2(sqa, sqb)
                    mn = su * (1.0 / HID)
                    var = qu * (1.0 / HID) - mn * mn
                    rs = _rsqrt(var + 1e-12)
                    m_a, m_b = _bcast(mn, 0), _bcast(mn, 8)
                    r_a, r_b = _bcast(rs, 0), _bcast(rs, 8)
                    for kk in range(KV):
                        wr[ta, pl.ds(kk * L16, L16)] = (
                            (acc_a[kk] - m_a) * (r_a * gvec[kk]) + bvec[kk])
                        wr[tb, pl.ds(kk * L16, L16)] = (
                            (acc_b[kk] - m_b) * (r_b * gvec[kk]) + bvec[kk])

            del _grp

        def do_chunk(g, p, p3, steady):
            wait_word(g, p)
            compute(g, p)
            issue_flush(g, p)
            if steady:
                @pl.when(jnp.logical_and(g >= 1, g + 3 < nchunk))
                def _():
                    wait_flush(p3)

                @pl.when(g + 3 < nchunk)
                def _():
                    issue_word(g + 3, p3)

        issue_word(0, 0)
        issue_word(1, 1)
        issue_word(2, 2)

        @pl.loop(0, (nchunk - 2) // NB)
        def _quad(i):
            g = i * NB
            do_chunk(g, 0, 3, True)
            do_chunk(g + 1, 1, 0, True)
            do_chunk(g + 2, 2, 1, True)
            do_chunk(g + 3, 3, 2, True)

        do_chunk(nchunk - 2, 0, 3, False)
        do_chunk(nchunk - 1, 1, 3, False)

        wait_flush(2)
        wait_flush(3)
        wait_flush(0)
        wait_flush(1)

    return k(idw2, ids5, wtab, dtab_f, ptab_f, stab_f, gamma, beta)


def kernel(word_ids, age_ids, bmi_ids, cycle_len_ids, seg_ids, posi_ids,
           word_table, demo_table, posi_table, seg_table, ln_gamma, ln_beta):
    b, l = word_ids.shape
    n_tok = b * l
    rows_w = n_tok // (NW * C)
    idw2 = word_ids.reshape(NW, 2 * rows_w, C // 2).astype(jnp.int32)
    as_w = lambda x: x.reshape(NW, rows_w, C).astype(jnp.int32)
    ids5 = jnp.stack(
        [as_w(age_ids) * HID, as_w(bmi_ids) * HID,
         as_w(cycle_len_ids) * HID, as_w(seg_ids), as_w(posi_ids) * HID],
        axis=1).reshape(NW, NS5 * rows_w, C)
    out = _embed_ln(
        n_tok, idw2, ids5,
        word_table.astype(jnp.float32),
        demo_table.astype(jnp.float32).reshape(-1),
        posi_table.astype(jnp.float32).reshape(-1),
        seg_table.astype(jnp.float32).reshape(-1),
        ln_gamma.astype(jnp.float32), ln_beta.astype(jnp.float32),
    )
    return out.reshape(b, l, HID)
